# Initial kernel scaffold; baseline (speedup 1.0000x reference)
import jax
import jax.numpy as jnp
from jax.experimental import pallas as pl


def kernel(x, Wr, br, W1, b1, W2, b2):
    raise NotImplementedError("write your pallas kernel here")


# speedup vs baseline: 1.3143x; 1.3143x over previous
import functools

import jax
import jax.numpy as jnp
from jax import lax
from jax.experimental import pallas as pl
from jax.experimental.pallas import tpu as pltpu
from jax.experimental.pallas import tpu_sc as plsc


def _router_body(C, E, TB, x_ref, wr_ref, br_ref,
                 ds_ref, dg_ref, kept_ref, p_ref, carry_ref):
    b = pl.program_id(0)

    @pl.when(b == 0)
    def _():
        carry_ref[...] = jnp.zeros_like(carry_ref)

    x = x_ref[...]
    wr = wr_ref[...]
    logits = lax.dot_general(x, wr, (((1,), (1,)), ((), ())),
                             preferred_element_type=jnp.float32)
    logits = logits + br_ref[...]
    lmax = jnp.max(logits, axis=1, keepdims=True)
    denom = jnp.sum(jnp.exp(logits - lmax), axis=1, keepdims=True)
    p = 1.0 / denom

    iota_e = lax.broadcasted_iota(jnp.int32, (TB, E), 1)
    is_max = logits == lmax
    route = jnp.min(jnp.where(is_max, iota_e, E), axis=1, keepdims=True)
    onehot = (iota_e == route).astype(jnp.float32)

    r = lax.broadcasted_iota(jnp.int32, (TB, TB), 0)
    c = lax.broadcasted_iota(jnp.int32, (TB, TB), 1)
    tri = (r >= c).astype(jnp.float32)
    pos_incl = lax.dot_general(tri, onehot, (((1,), (0,)), ((), ())),
                               preferred_element_type=jnp.float32)

    tot = pos_incl + carry_ref[...]
    slot1 = jnp.sum(onehot * tot, axis=1, keepdims=True)
    slot0 = slot1.astype(jnp.int32) - 1
    kept = slot0 < C
    dest = route * C + slot0
    ds_ref[...] = jnp.where(kept, dest, E * C)
    dg_ref[...] = jnp.where(kept, dest, 0)
    kept_ref[...] = kept.astype(jnp.float32)
    p_ref[...] = p
    carry_ref[...] = carry_ref[...] + jnp.sum(onehot, axis=0, keepdims=True)


def _router(xf, Wr, br, C, TB=512):
    T, D = xf.shape
    E = Wr.shape[0]
    nb = T // TB
    out = jax.ShapeDtypeStruct
    return pl.pallas_call(
        functools.partial(_router_body, C, E, TB),
        grid=(nb,),
        in_specs=[
            pl.BlockSpec((TB, D), lambda b: (b, 0)),
            pl.BlockSpec((E, D), lambda b: (0, 0)),
            pl.BlockSpec((1, E), lambda b: (0, 0)),
        ],
        out_specs=[
            pl.BlockSpec((TB, 1), lambda b: (b, 0)),
            pl.BlockSpec((TB, 1), lambda b: (b, 0)),
            pl.BlockSpec((TB, 1), lambda b: (b, 0)),
            pl.BlockSpec((TB, 1), lambda b: (b, 0)),
        ],
        out_shape=[
            out((T, 1), jnp.int32),
            out((T, 1), jnp.int32),
            out((T, 1), jnp.float32),
            out((T, 1), jnp.float32),
        ],
        scratch_shapes=[pltpu.VMEM((1, E), jnp.float32)],
    )(xf, Wr, br.reshape(1, E))


def _make_dispatch(T, D, n_rows):
    info = plsc.get_sparse_core_info()
    nw = info.num_cores * info.num_subcores
    per_w = T // nw
    ch = min(per_w, 64)
    nch = per_w // ch
    mesh = plsc.VectorSubcoreMesh(core_axis_name="c", subcore_axis_name="s")

    @functools.partial(
        pl.kernel,
        out_type=jax.ShapeDtypeStruct((n_rows, D), jnp.float32),
        mesh=mesh,
        scratch_types=[
            pltpu.VMEM((ch,), jnp.int32),
            pltpu.VMEM((ch, D), jnp.float32),
            pltpu.SemaphoreType.DMA,
        ],
    )
    def dispatch(xf_hbm, dest_hbm, out_hbm, idx_v, rows_v, sem):
        wid = lax.axis_index("s") * info.num_cores + lax.axis_index("c")
        base = wid * per_w
        for j in range(nch):
            b = base + j * ch
            pltpu.sync_copy(dest_hbm.at[pl.ds(b, ch)], idx_v)
            pltpu.sync_copy(xf_hbm.at[pl.ds(b, ch)], rows_v)
            pltpu.async_copy(rows_v, out_hbm.at[idx_v], sem).wait()

    return dispatch


def _make_combine(T, D):
    info = plsc.get_sparse_core_info()
    nw = info.num_cores * info.num_subcores
    per_w = T // nw
    ch = min(per_w, 64)
    nch = per_w // ch
    mesh = plsc.VectorSubcoreMesh(core_axis_name="c", subcore_axis_name="s")

    @functools.partial(
        pl.kernel,
        out_type=jax.ShapeDtypeStruct((T, D), jnp.float32),
        mesh=mesh,
        scratch_types=[
            pltpu.VMEM((ch,), jnp.int32),
            pltpu.VMEM((ch, D), jnp.float32),
            pltpu.SemaphoreType.DMA,
        ],
    )
    def combine(y_hbm, dest_hbm, out_hbm, idx_v, rows_v, sem):
        wid = lax.axis_index("s") * info.num_cores + lax.axis_index("c")
        base = wid * per_w
        for j in range(nch):
            b = base + j * ch
            pltpu.sync_copy(dest_hbm.at[pl.ds(b, ch)], idx_v)
            pltpu.async_copy(y_hbm.at[idx_v], rows_v, sem).wait()
            pltpu.sync_copy(rows_v, out_hbm.at[pl.ds(b, ch)])

    return combine


def _ffn_body(C, D, FB, x_ref, w1_ref, b1_ref, w2_ref, b2_ref, y_ref):
    f = pl.program_id(1)

    @pl.when(f == 0)
    def _():
        y_ref[...] = jnp.broadcast_to(b2_ref[...][0], (C, D))

    x = x_ref[...]
    w1 = w1_ref[...][0]
    h = lax.dot_general(x, w1, (((1,), (0,)), ((), ())),
                        preferred_element_type=jnp.float32)
    h = jnp.maximum(h + b1_ref[...][0], 0.0)
    w2 = w2_ref[...][0]
    y_ref[...] += lax.dot_general(h, w2, (((1,), (0,)), ((), ())),
                                  preferred_element_type=jnp.float32)


def _ffn(ein, W1, b1, W2, b2, C, FB=512):
    E, D, F = W1.shape
    nf = F // FB
    return pl.pallas_call(
        functools.partial(_ffn_body, C, D, FB),
        grid=(E, nf),
        in_specs=[
            pl.BlockSpec((C, D), lambda e, f: (e, 0)),
            pl.BlockSpec((1, D, FB), lambda e, f: (e, 0, f)),
            pl.BlockSpec((1, 1, FB), lambda e, f: (e, 0, f)),
            pl.BlockSpec((1, FB, D), lambda e, f: (e, f, 0)),
            pl.BlockSpec((1, 1, D), lambda e, f: (e, 0, 0)),
        ],
        out_specs=pl.BlockSpec((C, D), lambda e, f: (e, 0)),
        out_shape=jax.ShapeDtypeStruct((E * C, D), jnp.float32),
        compiler_params=pltpu.CompilerParams(
            vmem_limit_bytes=100 * 1024 * 1024),
    )(ein, W1, b1.reshape(E, 1, F), W2, b2.reshape(E, 1, D))


def _blend_body(g_ref, x_ref, kept_ref, p_ref, o_ref):
    o_ref[...] = jnp.where(kept_ref[...] > 0.0, g_ref[...], x_ref[...]) \
        * p_ref[...]


def _blend(g, xf, kept, p, TB=512):
    T, D = xf.shape
    nb = T // TB
    return pl.pallas_call(
        _blend_body,
        grid=(nb,),
        in_specs=[
            pl.BlockSpec((TB, D), lambda b: (b, 0)),
            pl.BlockSpec((TB, D), lambda b: (b, 0)),
            pl.BlockSpec((TB, 1), lambda b: (b, 0)),
            pl.BlockSpec((TB, 1), lambda b: (b, 0)),
        ],
        out_specs=pl.BlockSpec((TB, D), lambda b: (b, 0)),
        out_shape=jax.ShapeDtypeStruct((T, D), jnp.float32),
    )(g, xf, kept, p)


def kernel(x, Wr, br, W1, b1, W2, b2):
    seq, bsz, D = x.shape
    T = seq * bsz
    E = Wr.shape[0]
    C = int(1.25 * T / E)
    xf = x.reshape(T, D)

    dest_s, dest_g, kept, p = _router(xf, Wr, br, C)
    dest_s = dest_s.reshape(T)
    dest_g = dest_g.reshape(T)

    ein = _make_dispatch(T, D, E * C + 8)(xf, dest_s)
    y = _ffn(ein, W1, b1, W2, b2, C)
    g = _make_combine(T, D)(y, dest_g)
    out = _blend(g, xf, kept, p)
    return out.reshape(seq, bsz, D)
